# baseline (speedup 1.0000x reference)
import functools

import jax
import jax.numpy as jnp
from jax import lax
from jax.experimental import pallas as pl
from jax.experimental.pallas import tpu as pltpu
from jax.experimental.pallas import tpu_sc as plsc

_NQ = 128
_L = 16


def _sel_body(intra_ref, scores_ref, x_ref, newX_ref, idx_ref, gidx_ref):
    ns = intra_ref.shape[-1]
    n = scores_ref.shape[-1]
    kc = ns // 2
    m = kc + _NQ
    bb = pl.program_id(0)

    it_row = intra_ref[0]
    s_row = scores_ref[0]
    it_col = it_row.reshape(ns, 1)
    j_col = jax.lax.broadcasted_iota(jnp.int32, (ns, 1), 0)
    j_row = jax.lax.broadcasted_iota(jnp.int32, (1, ns), 1)

    CH = 384
    rank_chunks = []
    for c0 in range(0, ns, CH):
        it_i = jax.lax.slice(it_row, (0, c0), (1, c0 + CH))
        i_row = jax.lax.broadcasted_iota(jnp.int32, (1, CH), 1) + c0
        less = it_col < it_i
        tie = (it_col == it_i) & (j_col < i_row)
        mask = (less | tie).astype(jnp.int32)
        rank_chunks.append(jnp.sum(mask, axis=0, keepdims=True))
    rank_row = jnp.concatenate(rank_chunks, axis=1)

    s_supp_row = jax.lax.slice(s_row, (0, 0), (1, ns))
    PCH = 192
    idx_chunks, val_chunks = [], []
    for p0 in range(0, kc, PCH):
        p_col = jax.lax.broadcasted_iota(jnp.int32, (PCH, 1), 0) + p0
        onehot = rank_row == p_col
        idx_chunks.append(jnp.sum(
            jnp.where(onehot, j_row, 0), axis=1, keepdims=True))
        val_chunks.append(jnp.sum(
            jnp.where(onehot, s_supp_row, 0.0), axis=1, keepdims=True))
    q_iota = jax.lax.broadcasted_iota(jnp.int32, (_NQ, 1), 0) + ns
    s_col = s_row.reshape(n, 1)
    idx_col = jnp.concatenate(idx_chunks + [q_iota], axis=0)
    val_col = jnp.concatenate(
        val_chunks + [jax.lax.slice(s_col, (ns, 0), (n, 1))], axis=0)
    idx_ref[0] = idx_col.reshape(1, m)
    gidx_ref[0] = (idx_col + bb * n).reshape(1, m)

    jn_row = jax.lax.broadcasted_iota(jnp.int32, (1, n), 1)
    RCH = 272
    for r0 in range(0, m, RCH):
        idx_c = jax.lax.slice(idx_col, (r0, 0), (r0 + RCH, 1))
        val_c = jax.lax.slice(val_col, (r0, 0), (r0 + RCH, 1))
        g_c = (idx_c == jn_row).astype(jnp.float32)
        newX_ref[0, pl.ds(r0, RCH), :] = jax.lax.dot_general(
            g_c * val_c, x_ref[0], (((1,), (0,)), ((), ())),
            preferred_element_type=jnp.float32)


def _select(intra, scores, X):
    B, N, D = X.shape
    ns = N - _NQ
    m = ns // 2 + _NQ
    return pl.pallas_call(
        _sel_body,
        grid=(B,),
        in_specs=[
            pl.BlockSpec((1, 1, ns), lambda b_: (b_, 0, 0)),
            pl.BlockSpec((1, 1, N), lambda b_: (b_, 0, 0)),
            pl.BlockSpec((1, N, D), lambda b_: (b_, 0, 0)),
        ],
        out_specs=[
            pl.BlockSpec((1, m, D), lambda b_: (b_, 0, 0)),
            pl.BlockSpec((1, 1, m), lambda b_: (b_, 0, 0)),
            pl.BlockSpec((1, 1, m), lambda b_: (b_, 0, 0)),
        ],
        out_shape=[
            jax.ShapeDtypeStruct((B, m, D), jnp.float32),
            jax.ShapeDtypeStruct((B, 1, m), jnp.int32),
            jax.ShapeDtypeStruct((B, 1, m), jnp.int32),
        ],
        compiler_params=pltpu.CompilerParams(
            dimension_semantics=("arbitrary",)),
    )(intra.reshape(B, 1, ns), scores.reshape(B, 1, N), X)


def _sc_gather(A2, gidx, colidx):
    BN, N = A2.shape
    B, m = colidx.shape
    R = B * m
    info = plsc.get_sparse_core_info()
    NC, NS = info.num_cores, info.num_subcores
    NW = NC * NS
    WPB = NW // B
    NROW = m // WPB
    G = 8
    NCH = NROW // G
    NV = m // _L

    mesh = plsc.VectorSubcoreMesh(core_axis_name="c", subcore_axis_name="s")

    @functools.partial(
        pl.kernel, mesh=mesh,
        out_type=jax.ShapeDtypeStruct((R * m,), jnp.float32),
        scratch_types=[
            pltpu.VMEM((m,), jnp.int32),
            pltpu.VMEM((NROW,), jnp.int32),
            pltpu.VMEM((G, N), jnp.float32),
            pltpu.VMEM((G * m,), jnp.float32),
            pltpu.SemaphoreType.DMA,
        ],
        compiler_params=pltpu.CompilerParams(needs_layout_passes=False),
    )
    def k(a2, gidx_h, cidx_h, out, colv, rowv, rbuf, obuf, sem):
        wid = lax.axis_index("s") * NC + lax.axis_index("c")
        b = wid // WPB
        pltpu.sync_copy(cidx_h.at[b], colv)
        pltpu.sync_copy(gidx_h.at[pl.ds(wid * NROW, NROW)], rowv)

        def chunk(ci, carry):
            pltpu.async_copy(a2.at[rowv.at[pl.ds(ci * G, G)]], rbuf,
                             sem).wait()

            def vstep(v, c2):
                cvec = colv[pl.ds(v * _L, _L)]
                for i in range(G):
                    ri = jnp.full((_L,), i, jnp.int32)
                    vals = plsc.load_gather(rbuf, [ri, cvec])
                    obuf[pl.ds(i * m + v * _L, _L)] = vals
                return c2

            lax.fori_loop(0, NV, vstep, 0)
            pltpu.sync_copy(
                obuf, out.at[pl.ds((wid * NROW + ci * G) * m, G * m)])
            return carry

        lax.fori_loop(0, NCH, chunk, 0)

    return k(A2, gidx, colidx)


def kernel(A, X, W, b):
    B, N, D = X.shape
    ns = N - _NQ
    m = ns // 2 + _NQ
    scores = jax.nn.sigmoid(jnp.squeeze(X @ W + b, -1) / 100.0)
    supp = scores[:, :ns]
    intra = supp - jnp.mean(supp, axis=1, keepdims=True)

    newX, idx3, gidx3 = _select(intra, scores, X)
    newA_flat = _sc_gather(A.reshape(B * N, N), gidx3.reshape(B * m),
                           idx3.reshape(B, m))
    return newA_flat.reshape(B, m, m), newX, idx3.reshape(B, m)


# speedup vs baseline: 1.2303x; 1.2303x over previous
import functools

import jax
import jax.numpy as jnp
from jax import lax
from jax.experimental import pallas as pl
from jax.experimental.pallas import tpu as pltpu
from jax.experimental.pallas import tpu_sc as plsc

_NQ = 128
_L = 16


def _sel_body(intra_ref, scores_ref, x_ref, newX_ref, idx_ref, gidx_ref):
    ns = intra_ref.shape[-1]
    n = scores_ref.shape[-1]
    kc = ns // 2
    m = kc + _NQ
    bb = pl.program_id(0)

    it_row = intra_ref[0]
    s_row = scores_ref[0]
    it_col = it_row.reshape(ns, 1)
    j_col = jax.lax.broadcasted_iota(jnp.int32, (ns, 1), 0)
    j_row = jax.lax.broadcasted_iota(jnp.int32, (1, ns), 1)

    CH = 384
    rank_chunks = []
    for c0 in range(0, ns, CH):
        it_i = jax.lax.slice(it_row, (0, c0), (1, c0 + CH))
        i_row = jax.lax.broadcasted_iota(jnp.int32, (1, CH), 1) + c0
        less = it_col < it_i
        tie = (it_col == it_i) & (j_col < i_row)
        mask = (less | tie).astype(jnp.int32)
        rank_chunks.append(jnp.sum(mask, axis=0, keepdims=True))
    rank_row = jnp.concatenate(rank_chunks, axis=1)

    s_supp_row = jax.lax.slice(s_row, (0, 0), (1, ns))
    PCH = 192
    idx_chunks, val_chunks = [], []
    for p0 in range(0, kc, PCH):
        p_col = jax.lax.broadcasted_iota(jnp.int32, (PCH, 1), 0) + p0
        onehot = rank_row == p_col
        idx_chunks.append(jnp.sum(
            jnp.where(onehot, j_row, 0), axis=1, keepdims=True))
        val_chunks.append(jnp.sum(
            jnp.where(onehot, s_supp_row, 0.0), axis=1, keepdims=True))
    q_iota = jax.lax.broadcasted_iota(jnp.int32, (_NQ, 1), 0) + ns
    s_col = s_row.reshape(n, 1)
    idx_col = jnp.concatenate(idx_chunks + [q_iota], axis=0)
    val_col = jnp.concatenate(
        val_chunks + [jax.lax.slice(s_col, (ns, 0), (n, 1))], axis=0)
    idx_ref[0] = idx_col.reshape(1, m)
    gidx_ref[0] = (idx_col + bb * n).reshape(1, m)

    jn_row = jax.lax.broadcasted_iota(jnp.int32, (1, n), 1)
    RCH = 272
    for r0 in range(0, m, RCH):
        idx_c = jax.lax.slice(idx_col, (r0, 0), (r0 + RCH, 1))
        val_c = jax.lax.slice(val_col, (r0, 0), (r0 + RCH, 1))
        g_c = (idx_c == jn_row).astype(jnp.float32)
        newX_ref[0, pl.ds(r0, RCH), :] = jax.lax.dot_general(
            g_c * val_c, x_ref[0], (((1,), (0,)), ((), ())),
            preferred_element_type=jnp.float32)


def _select(intra, scores, X):
    B, N, D = X.shape
    ns = N - _NQ
    m = ns // 2 + _NQ
    return pl.pallas_call(
        _sel_body,
        grid=(B,),
        in_specs=[
            pl.BlockSpec((1, 1, ns), lambda b_: (b_, 0, 0)),
            pl.BlockSpec((1, 1, N), lambda b_: (b_, 0, 0)),
            pl.BlockSpec((1, N, D), lambda b_: (b_, 0, 0)),
        ],
        out_specs=[
            pl.BlockSpec((1, m, D), lambda b_: (b_, 0, 0)),
            pl.BlockSpec((1, 1, m), lambda b_: (b_, 0, 0)),
            pl.BlockSpec((1, 1, m), lambda b_: (b_, 0, 0)),
        ],
        out_shape=[
            jax.ShapeDtypeStruct((B, m, D), jnp.float32),
            jax.ShapeDtypeStruct((B, 1, m), jnp.int32),
            jax.ShapeDtypeStruct((B, 1, m), jnp.int32),
        ],
        compiler_params=pltpu.CompilerParams(
            dimension_semantics=("arbitrary",)),
    )(intra.reshape(B, 1, ns), scores.reshape(B, 1, N), X)


def _sc_gather(A2, gidx, colidx):
    BN, N = A2.shape
    B, m = colidx.shape
    R = B * m
    info = plsc.get_sparse_core_info()
    NC, NS = info.num_cores, info.num_subcores
    NW = NC * NS
    WPB = NW // B
    NROW = m // WPB
    G = 8
    NCH = NROW // G
    NV = m // _L

    HMAX = (NCH - 1) // 2
    mesh = plsc.VectorSubcoreMesh(core_axis_name="c", subcore_axis_name="s")

    @functools.partial(
        pl.kernel, mesh=mesh,
        out_type=jax.ShapeDtypeStruct((R * m,), jnp.float32),
        scratch_types=[
            pltpu.VMEM((m,), jnp.int32),
            pltpu.VMEM((NROW,), jnp.int32),
            pltpu.VMEM((G, N), jnp.float32),
            pltpu.VMEM((G, N), jnp.float32),
            pltpu.VMEM((G * m,), jnp.float32),
            pltpu.VMEM((G * m,), jnp.float32),
            pltpu.SemaphoreType.DMA,
            pltpu.SemaphoreType.DMA,
            pltpu.SemaphoreType.DMA,
            pltpu.SemaphoreType.DMA,
        ],
        compiler_params=pltpu.CompilerParams(needs_layout_passes=False),
    )
    def k(a2, gidx_h, cidx_h, out, colv, rowv, rb0, rb1, ob0, ob1,
          sg0, sg1, sw0, sw1):
        wid = lax.axis_index("s") * NC + lax.axis_index("c")
        b = wid // WPB
        pltpu.sync_copy(cidx_h.at[b], colv)
        pltpu.sync_copy(gidx_h.at[pl.ds(wid * NROW, NROW)], rowv)

        def g_copy(ci, rb, sem):
            return pltpu.make_async_copy(
                a2.at[rowv.at[pl.ds(ci * G, G)]], rb, sem)

        def w_copy(ci, ob, sem):
            return pltpu.make_async_copy(
                ob, out.at[pl.ds((wid * NROW + ci * G) * m, G * m)], sem)

        def compute(rb, ob):
            def vstep(v, c2):
                cvec = colv[pl.ds(v * _L, _L)]
                for i in range(G):
                    ri = jnp.full((_L,), i, jnp.int32)
                    ob[pl.ds(i * m + v * _L, _L)] = plsc.load_gather(
                        rb, [ri, cvec])
                return c2

            lax.fori_loop(0, NV, vstep, 0)

        g_copy(0, rb0, sg0).start()
        g_copy(1, rb1, sg1).start()

        def body(h, carry):
            a = 2 * h

            @pl.when(h > 0)
            def _():
                w_copy(a - 2, ob0, sw0).wait()

            g_copy(a, rb0, sg0).wait()
            compute(rb0, ob0)
            g_copy(a + 2, rb0, sg0).start()
            w_copy(a, ob0, sw0).start()

            @pl.when(h > 0)
            def _():
                w_copy(a - 1, ob1, sw1).wait()

            g_copy(a + 1, rb1, sg1).wait()
            compute(rb1, ob1)

            @pl.when(h < HMAX - 1)
            def _():
                g_copy(a + 3, rb1, sg1).start()

            w_copy(a + 1, ob1, sw1).start()
            return carry

        lax.fori_loop(0, HMAX, body, 0)
        w_copy(NCH - 3, ob0, sw0).wait()
        g_copy(NCH - 1, rb0, sg0).wait()
        compute(rb0, ob0)
        w_copy(NCH - 1, ob0, sw0).start()
        w_copy(NCH - 1, ob0, sw0).wait()
        w_copy(NCH - 2, ob1, sw1).wait()

    return k(A2, gidx, colidx)


def kernel(A, X, W, b):
    B, N, D = X.shape
    ns = N - _NQ
    m = ns // 2 + _NQ
    scores = jax.nn.sigmoid(jnp.squeeze(X @ W + b, -1) / 100.0)
    supp = scores[:, :ns]
    intra = supp - jnp.mean(supp, axis=1, keepdims=True)

    newX, idx3, gidx3 = _select(intra, scores, X)
    newA_flat = _sc_gather(A.reshape(B * N, N), gidx3.reshape(B * m),
                           idx3.reshape(B, m))
    return newA_flat.reshape(B, m, m), newX, idx3.reshape(B, m)


# speedup vs baseline: 1.4965x; 1.2163x over previous
import functools

import jax
import jax.numpy as jnp
from jax import lax
from jax.experimental import pallas as pl
from jax.experimental.pallas import tpu as pltpu
from jax.experimental.pallas import tpu_sc as plsc

_NQ = 128
_L = 16


def _sel_body(intra_ref, scores_ref, x_ref, newX_ref, idx_ref, gidx_ref):
    ns = intra_ref.shape[-1]
    n = scores_ref.shape[-1]
    kc = ns // 2
    m = kc + _NQ
    bb = pl.program_id(0)

    it_row = intra_ref[0]
    s_row = scores_ref[0]
    it_col = it_row.reshape(ns, 1)
    j_col = jax.lax.broadcasted_iota(jnp.int32, (ns, 1), 0)
    j_row = jax.lax.broadcasted_iota(jnp.int32, (1, ns), 1)

    CH = 384
    rank_chunks = []
    for c0 in range(0, ns, CH):
        it_i = jax.lax.slice(it_row, (0, c0), (1, c0 + CH))
        i_row = jax.lax.broadcasted_iota(jnp.int32, (1, CH), 1) + c0
        less = it_col < it_i
        tie = (it_col == it_i) & (j_col < i_row)
        mask = (less | tie).astype(jnp.int32)
        rank_chunks.append(jnp.sum(mask, axis=0, keepdims=True))
    rank_row = jnp.concatenate(rank_chunks, axis=1)

    s_supp_row = jax.lax.slice(s_row, (0, 0), (1, ns))
    PCH = 192
    idx_chunks, val_chunks = [], []
    for p0 in range(0, kc, PCH):
        p_col = jax.lax.broadcasted_iota(jnp.int32, (PCH, 1), 0) + p0
        onehot = rank_row == p_col
        idx_chunks.append(jnp.sum(
            jnp.where(onehot, j_row, 0), axis=1, keepdims=True))
        val_chunks.append(jnp.sum(
            jnp.where(onehot, s_supp_row, 0.0), axis=1, keepdims=True))
    q_iota = jax.lax.broadcasted_iota(jnp.int32, (_NQ, 1), 0) + ns
    s_col = s_row.reshape(n, 1)
    idx_col = jnp.concatenate(idx_chunks + [q_iota], axis=0)
    val_col = jnp.concatenate(
        val_chunks + [jax.lax.slice(s_col, (ns, 0), (n, 1))], axis=0)
    idx_ref[0] = idx_col.reshape(1, m)
    gidx_ref[0] = (idx_col + bb * n).reshape(1, m)

    jn_row = jax.lax.broadcasted_iota(jnp.int32, (1, n), 1)
    RCH = 272
    for r0 in range(0, m, RCH):
        idx_c = jax.lax.slice(idx_col, (r0, 0), (r0 + RCH, 1))
        val_c = jax.lax.slice(val_col, (r0, 0), (r0 + RCH, 1))
        g_c = (idx_c == jn_row).astype(jnp.float32)
        newX_ref[0, pl.ds(r0, RCH), :] = jax.lax.dot_general(
            g_c * val_c, x_ref[0], (((1,), (0,)), ((), ())),
            preferred_element_type=jnp.float32)


def _select(intra, scores, X):
    B, N, D = X.shape
    ns = N - _NQ
    m = ns // 2 + _NQ
    return pl.pallas_call(
        _sel_body,
        grid=(B,),
        in_specs=[
            pl.BlockSpec((1, 1, ns), lambda b_: (b_, 0, 0)),
            pl.BlockSpec((1, 1, N), lambda b_: (b_, 0, 0)),
            pl.BlockSpec((1, N, D), lambda b_: (b_, 0, 0)),
        ],
        out_specs=[
            pl.BlockSpec((1, m, D), lambda b_: (b_, 0, 0)),
            pl.BlockSpec((1, 1, m), lambda b_: (b_, 0, 0)),
            pl.BlockSpec((1, 1, m), lambda b_: (b_, 0, 0)),
        ],
        out_shape=[
            jax.ShapeDtypeStruct((B, m, D), jnp.float32),
            jax.ShapeDtypeStruct((B, 1, m), jnp.int32),
            jax.ShapeDtypeStruct((B, 1, m), jnp.int32),
        ],
        compiler_params=pltpu.CompilerParams(
            dimension_semantics=("arbitrary",)),
    )(intra.reshape(B, 1, ns), scores.reshape(B, 1, N), X)


def _sc_gather(A2, gidx, colidx):
    BN, N = A2.shape
    B, m = colidx.shape
    R = B * m
    info = plsc.get_sparse_core_info()
    NC, NS = info.num_cores, info.num_subcores
    NW = NC * NS
    WPB = NW // B
    NROW = m // WPB
    G = 8
    NCH = NROW // G
    NV = m // _L

    HMAX = (NCH - 1) // 2
    mesh = plsc.VectorSubcoreMesh(core_axis_name="c", subcore_axis_name="s")

    @functools.partial(
        pl.kernel, mesh=mesh,
        out_type=jax.ShapeDtypeStruct((R, m), jnp.float32),
        scratch_types=[
            pltpu.VMEM((m,), jnp.int32),
            pltpu.VMEM((NROW,), jnp.int32),
            pltpu.VMEM((G, N), jnp.float32),
            pltpu.VMEM((G, N), jnp.float32),
            pltpu.VMEM((G, m), jnp.float32),
            pltpu.VMEM((G, m), jnp.float32),
            pltpu.SemaphoreType.DMA,
            pltpu.SemaphoreType.DMA,
            pltpu.SemaphoreType.DMA,
            pltpu.SemaphoreType.DMA,
        ],
        compiler_params=pltpu.CompilerParams(needs_layout_passes=False),
    )
    def k(a2, gidx_h, cidx_h, out, colv, rowv, rb0, rb1, ob0, ob1,
          sg0, sg1, sw0, sw1):
        wid = lax.axis_index("s") * NC + lax.axis_index("c")
        b = wid // WPB
        pltpu.sync_copy(cidx_h.at[b], colv)
        pltpu.sync_copy(gidx_h.at[pl.ds(wid * NROW, NROW)], rowv)

        def g_copy(ci, rb, sem):
            return pltpu.make_async_copy(
                a2.at[rowv.at[pl.ds(ci * G, G)]], rb, sem)

        def w_copy(ci, ob, sem):
            return pltpu.make_async_copy(
                ob, out.at[pl.ds(wid * NROW + ci * G, G)], sem)

        def compute(rb, ob):
            def vstep(v, c2):
                cvec = colv[pl.ds(v * _L, _L)]
                for i in range(G):
                    ri = jnp.full((_L,), i, jnp.int32)
                    ob[i, pl.ds(v * _L, _L)] = plsc.load_gather(
                        rb, [ri, cvec])
                return c2

            lax.fori_loop(0, NV, vstep, 0)

        g_copy(0, rb0, sg0).start()
        g_copy(1, rb1, sg1).start()

        def body(h, carry):
            a = 2 * h

            @pl.when(h > 0)
            def _():
                w_copy(a - 2, ob0, sw0).wait()

            g_copy(a, rb0, sg0).wait()
            compute(rb0, ob0)
            g_copy(a + 2, rb0, sg0).start()
            w_copy(a, ob0, sw0).start()

            @pl.when(h > 0)
            def _():
                w_copy(a - 1, ob1, sw1).wait()

            g_copy(a + 1, rb1, sg1).wait()
            compute(rb1, ob1)

            @pl.when(h < HMAX - 1)
            def _():
                g_copy(a + 3, rb1, sg1).start()

            w_copy(a + 1, ob1, sw1).start()
            return carry

        lax.fori_loop(0, HMAX, body, 0)
        w_copy(NCH - 3, ob0, sw0).wait()
        g_copy(NCH - 1, rb0, sg0).wait()
        compute(rb0, ob0)
        w_copy(NCH - 1, ob0, sw0).start()
        w_copy(NCH - 1, ob0, sw0).wait()
        w_copy(NCH - 2, ob1, sw1).wait()

    return k(A2, gidx, colidx)


def kernel(A, X, W, b):
    B, N, D = X.shape
    ns = N - _NQ
    m = ns // 2 + _NQ
    scores = jax.nn.sigmoid(jnp.squeeze(X @ W + b, -1) / 100.0)
    supp = scores[:, :ns]
    intra = supp - jnp.mean(supp, axis=1, keepdims=True)

    newX, idx3, gidx3 = _select(intra, scores, X)
    newA2 = _sc_gather(A.reshape(B * N, N), gidx3.reshape(B * m),
                       idx3.reshape(B, m))
    return newA2.reshape(B, m, m), newX, idx3.reshape(B, m)
